# Initial kernel scaffold; baseline (speedup 1.0000x reference)
import jax
import jax.numpy as jnp
from jax.experimental import pallas as pl


def kernel(in_pc, target_pc):
    raise NotImplementedError("write your pallas kernel here")


# speedup vs baseline: 233.0766x; 233.0766x over previous
import jax
import jax.numpy as jnp
from jax.experimental import pallas as pl
from jax.experimental.pallas import tpu as pltpu

TILE = 1024


def _chamfer_body(p1t_ref, p2_ref, loss_ref, colmin_ref):
    b = pl.program_id(0)
    i = pl.program_id(1)
    nt = pl.num_programs(1)

    p = p1t_ref[0]
    q = p2_ref[0]

    pb = p.astype(jnp.bfloat16).astype(jnp.float32)
    qb = q.astype(jnp.bfloat16).astype(jnp.float32)
    cross = (pb[:, 0:1] * qb[0:1, :]
             + pb[:, 1:2] * qb[1:2, :]
             + pb[:, 2:3] * qb[2:3, :])
    pn = (p[:, 0:1] * p[:, 0:1]
          + p[:, 1:2] * p[:, 1:2]
          + p[:, 2:3] * p[:, 2:3])
    qn = (q[0:1, :] * q[0:1, :]
          + q[1:2, :] * q[1:2, :]
          + q[2:3, :] * q[2:3, :])
    d = ((-2.0) * cross + pn) + qn

    row_min = jnp.min(d, axis=1)
    col_min = jnp.min(d, axis=0)

    prev_col = colmin_ref[0, 0, :]
    new_col = jnp.where(i == 0, col_min, jnp.minimum(prev_col, col_min))
    colmin_ref[0, 0, :] = new_col

    acc = jnp.where((b == 0) & (i == 0), 0.0, loss_ref[0, 0])
    acc = acc + jnp.sum(row_min)
    acc = acc + jnp.where(i == nt - 1, jnp.sum(new_col), 0.0)
    loss_ref[0, 0] = acc


def kernel(in_pc, target_pc):
    B, C, N = in_pc.shape
    M = target_pc.shape[2]
    nt = N // TILE

    pc1_t = jnp.transpose(in_pc, (0, 2, 1))

    loss, _ = pl.pallas_call(
        _chamfer_body,
        grid=(B, nt),
        in_specs=[
            pl.BlockSpec((1, TILE, C), lambda b, i: (b, i, 0)),
            pl.BlockSpec((1, C, M), lambda b, i: (b, 0, 0)),
        ],
        out_specs=[
            pl.BlockSpec((1, 1), lambda b, i: (0, 0),
                         memory_space=pltpu.SMEM),
            pl.BlockSpec((1, 1, M), lambda b, i: (b, 0, 0)),
        ],
        out_shape=[
            jax.ShapeDtypeStruct((1, 1), jnp.float32),
            jax.ShapeDtypeStruct((B, 1, M), jnp.float32),
        ],
    )(pc1_t, target_pc)

    return loss[0, 0] / (2.0 * B * N)


# speedup vs baseline: 426.5129x; 1.8299x over previous
import jax
import jax.numpy as jnp
from jax.experimental import pallas as pl
from jax.experimental.pallas import tpu as pltpu

TILE = 1024


def _chamfer_body(p1t_ref, p2_ref, loss_ref, colmin_ref):
    b = pl.program_id(0)
    i = pl.program_id(1)
    nt = pl.num_programs(1)

    p = p1t_ref[0]
    q = p2_ref[0]

    pb = p.astype(jnp.bfloat16)
    qb = q.astype(jnp.bfloat16)
    cross = jax.lax.dot_general(
        pb, qb, (((1,), (0,)), ((), ())),
        preferred_element_type=jnp.float32)
    pn = (p[:, 0:1] * p[:, 0:1]
          + p[:, 1:2] * p[:, 1:2]
          + p[:, 2:3] * p[:, 2:3])
    qn = (q[0:1, :] * q[0:1, :]
          + q[1:2, :] * q[1:2, :]
          + q[2:3, :] * q[2:3, :])
    d = ((-2.0) * cross + pn) + qn

    row_min = jnp.min(d, axis=1)
    col_min = jnp.min(d, axis=0)

    prev_col = colmin_ref[0, 0, :]
    new_col = jnp.where(i == 0, col_min, jnp.minimum(prev_col, col_min))
    colmin_ref[0, 0, :] = new_col

    acc = jnp.where((b == 0) & (i == 0), 0.0, loss_ref[0, 0])
    acc = acc + jnp.sum(row_min)
    acc = acc + jnp.where(i == nt - 1, jnp.sum(new_col), 0.0)
    loss_ref[0, 0] = acc


def kernel(in_pc, target_pc):
    B, C, N = in_pc.shape
    M = target_pc.shape[2]
    nt = N // TILE

    pc1_t = jnp.transpose(in_pc, (0, 2, 1))

    loss, _ = pl.pallas_call(
        _chamfer_body,
        grid=(B, nt),
        in_specs=[
            pl.BlockSpec((1, TILE, C), lambda b, i: (b, i, 0)),
            pl.BlockSpec((1, C, M), lambda b, i: (b, 0, 0)),
        ],
        out_specs=[
            pl.BlockSpec((1, 1), lambda b, i: (0, 0),
                         memory_space=pltpu.SMEM),
            pl.BlockSpec((1, 1, M), lambda b, i: (b, 0, 0)),
        ],
        out_shape=[
            jax.ShapeDtypeStruct((1, 1), jnp.float32),
            jax.ShapeDtypeStruct((B, 1, M), jnp.float32),
        ],
    )(pc1_t, target_pc)

    return loss[0, 0] / (2.0 * B * N)
